# baseline (speedup 1.0000x reference)
import dataclasses
import functools

import jax
import jax.numpy as jnp
from jax import lax
from jax.experimental import pallas as pl
from jax.experimental.pallas import tpu as pltpu
from jax.experimental.pallas import tpu_sc as plsc

_EPS = 1e-15
_N = 4096
_E = 65536
_NC = 2
_NS = 16
_L = 16
_B = 128
_PER_TILE = _E // (_NC * _NS)
_NBLK = _PER_TILE // _B
_STRIPE = _N // _NS
_ROWS_PER_TILE = _PER_TILE // _B


def _mesh():
    return plsc.VectorSubcoreMesh(core_axis_name="c", subcore_axis_name="s")


def _zero_fill(buf, nrows, D):
    @pl.loop(0, nrows)
    def _(i):
        for k in range(D // _L):
            buf[i, pl.ds(k * _L, _L)] = jnp.zeros((_L,), jnp.float32)


def _init_acc(acc_sh, zbuf, sid, D):
    _zero_fill(zbuf, _B, D)
    for r in range(_STRIPE // _B):
        pltpu.sync_copy(zbuf, acc_sh.at[pl.ds(sid * _STRIPE + r * _B, _B)])


def _drain_acc(acc_sh, out_hbm, cid, sid):
    for r in range(_STRIPE // _B):
        off = sid * _STRIPE + r * _B
        pltpu.sync_copy(acc_sh.at[pl.ds(off, _B)], out_hbm.at[cid, pl.ds(off, _B)])


@functools.cache
def _make_sc_segsum():
    D = 128

    @functools.partial(
        pl.kernel,
        out_type=jax.ShapeDtypeStruct((_NC, _N, D), jnp.float32),
        mesh=_mesh(),
        scratch_types=[
            pltpu.VMEM((_ROWS_PER_TILE, _B), jnp.int32),
            pltpu.VMEM((_ROWS_PER_TILE, _B), jnp.int32),
            pltpu.VMEM((2, _B, _L), jnp.float32),
            pltpu.VMEM((2, _B, D), jnp.float32),
            pltpu.VMEM_SHARED((_N, D), jnp.float32),
            pltpu.SemaphoreType.DMA,
            pltpu.SemaphoreType.DMA,
            pltpu.SemaphoreType.DMA,
            pltpu.SemaphoreType.DMA,
        ],
    )
    def seg(table_hbm, gidx_hbm, sidx_hbm, wb_hbm, out_hbm,
            gidx_v, sidx_v, wb_v, rows_v, acc_sh, g0, g1, s0, s1):
        cid = lax.axis_index("c")
        sid = lax.axis_index("s")
        _init_acc(acc_sh, rows_v.at[0], sid, D)

        tile_base = (cid * _NS + sid) * _PER_TILE
        row_base = (cid * _NS + sid) * _ROWS_PER_TILE
        pltpu.sync_copy(gidx_hbm.at[pl.ds(row_base, _ROWS_PER_TILE)], gidx_v)
        pltpu.sync_copy(sidx_hbm.at[pl.ds(row_base, _ROWS_PER_TILE)], sidx_v)
        plsc.subcore_barrier()

        @pl.loop(0, _NBLK, step=2)
        def _(blk):
            ga = pltpu.async_copy(
                table_hbm.at[gidx_v.at[blk]], rows_v.at[0], g0)
            gb = pltpu.async_copy(
                table_hbm.at[gidx_v.at[blk + 1]], rows_v.at[1], g1)
            pltpu.sync_copy(wb_hbm.at[pl.ds(tile_base + blk * _B, _B)],
                            wb_v.at[0])
            pltpu.sync_copy(wb_hbm.at[pl.ds(tile_base + (blk + 1) * _B, _B)],
                            wb_v.at[1])
            scatters = []
            for b, gwait, ssem in ((0, ga, s0), (1, gb, s1)):
                gwait.wait()

                @plsc.parallel_loop(0, _B, unroll=4)
                def _(j):
                    c = wb_v[b, j, :]
                    for k in range(D // _L):
                        sl = (b, j, pl.ds(k * _L, _L))
                        rows_v[sl] = rows_v[sl] * c

                scatters.append(pltpu.async_copy(
                    rows_v.at[b], acc_sh.at[sidx_v.at[blk + b]], ssem,
                    add=True))
            for sc in scatters:
                sc.wait()

        plsc.subcore_barrier()
        _drain_acc(acc_sh, out_hbm, cid, sid)

    return seg


@functools.cache
def _make_sc_tfused():
    D = 128

    @functools.partial(
        pl.kernel,
        out_type=[jax.ShapeDtypeStruct((_NC, _N, D), jnp.float32)] * 2,
        mesh=_mesh(),
        scratch_types=[
            pltpu.VMEM((_ROWS_PER_TILE, _B), jnp.int32),
            pltpu.VMEM((_ROWS_PER_TILE, _B), jnp.int32),
            pltpu.VMEM((_B, _L), jnp.float32),
            pltpu.VMEM((_B, D), jnp.float32),
            pltpu.VMEM((_B, D), jnp.float32),
            pltpu.VMEM_SHARED((_N, D), jnp.float32),
            pltpu.VMEM_SHARED((_N, D), jnp.float32),
            pltpu.SemaphoreType.DMA,
            pltpu.SemaphoreType.DMA,
        ],
    )
    def seg2(tabl_hbm, tabr_hbm, gidx_hbm, sidx_hbm, wb_hbm, outl_hbm, outr_hbm,
             gidx_v, sidx_v, wb_v, rowsl_v, rowsr_v, accl_sh, accr_sh,
             sl0, sr0):
        cid = lax.axis_index("c")
        sid = lax.axis_index("s")
        _init_acc(accl_sh, rowsl_v, sid, D)
        _init_acc(accr_sh, rowsr_v, sid, D)

        tile_base = (cid * _NS + sid) * _PER_TILE
        row_base = (cid * _NS + sid) * _ROWS_PER_TILE
        pltpu.sync_copy(gidx_hbm.at[pl.ds(row_base, _ROWS_PER_TILE)], gidx_v)
        pltpu.sync_copy(sidx_hbm.at[pl.ds(row_base, _ROWS_PER_TILE)], sidx_v)
        plsc.subcore_barrier()

        @pl.loop(0, _NBLK)
        def _(blk):
            pltpu.sync_copy(wb_hbm.at[pl.ds(tile_base + blk * _B, _B)], wb_v)
            cpl = pltpu.async_copy(
                tabl_hbm.at[gidx_v.at[blk]], rowsl_v, sl0)
            cpr = pltpu.async_copy(
                tabr_hbm.at[gidx_v.at[blk]], rowsr_v, sr0)
            cpl.wait()
            cpr.wait()

            @plsc.parallel_loop(0, _B, unroll=4)
            def _(j):
                c = wb_v[j, :]
                for k in range(D // _L):
                    sl = (j, pl.ds(k * _L, _L))
                    rowsl_v[sl] = rowsl_v[sl] * c
                    rowsr_v[sl] = rowsr_v[sl] * c

            sa = pltpu.async_copy(
                rowsl_v, accl_sh.at[sidx_v.at[blk]], sl0, add=True)
            sb = pltpu.async_copy(
                rowsr_v, accr_sh.at[sidx_v.at[blk]], sr0, add=True)
            sa.wait()
            sb.wait()

        plsc.subcore_barrier()
        _drain_acc(accl_sh, outl_hbm, cid, sid)
        _drain_acc(accr_sh, outr_hbm, cid, sid)

    return seg2


@functools.cache
def _make_sc_deg():
    cp = pltpu.CompilerParams()
    if "needs_layout_passes" in pltpu.CompilerParams.__dataclass_fields__:
        cp = dataclasses.replace(cp, needs_layout_passes=False)

    @functools.partial(
        pl.kernel,
        out_type=jax.ShapeDtypeStruct((_NC * _NS, _N), jnp.float32),
        mesh=_mesh(),
        scratch_types=[
            pltpu.VMEM((_ROWS_PER_TILE, _B), jnp.int32),
            pltpu.VMEM((_PER_TILE,), jnp.float32),
            pltpu.VMEM((_N,), jnp.float32),
        ],
        compiler_params=cp,
    )
    def degk(sidx_hbm, w_hbm, out_hbm, sidx_v, w_v, acc_v):
        cid = lax.axis_index("c")
        sid = lax.axis_index("s")
        wid = cid * _NS + sid

        for i in range(_N // _L):
            acc_v[pl.ds(i * _L, _L)] = jnp.zeros((_L,), jnp.float32)

        tile_base = wid * _PER_TILE
        row_base = wid * _ROWS_PER_TILE
        pltpu.sync_copy(sidx_hbm.at[pl.ds(row_base, _ROWS_PER_TILE)], sidx_v)
        pltpu.sync_copy(w_hbm.at[pl.ds(tile_base, _PER_TILE)], w_v)

        for blk in range(_NBLK):
            for g in range(_B // _L):
                idx16 = sidx_v[blk, pl.ds(g * _L, _L)]
                w16 = w_v[pl.ds((blk * (_B // _L) + g) * _L, _L)]
                plsc.addupdate_scatter(acc_v, [idx16], w16)

        pltpu.sync_copy(acc_v, out_hbm.at[wid])

    return degk


def _eye(C, dtype=jnp.float32):
    r = lax.broadcasted_iota(jnp.int32, (C, C), 0)
    c = lax.broadcasted_iota(jnp.int32, (C, C), 1)
    return jnp.where(r == c, 1.0, 0.0).astype(dtype)


def _tdot(a, b):
    return lax.dot_general(a, b, dimension_numbers=(((0,), (0,)), ((), ())),
                           preferred_element_type=jnp.float32)


def _dot(a, b):
    return jnp.dot(a, b, preferred_element_type=jnp.float32)


def _softmax(s):
    m = jnp.max(s, axis=-1, keepdims=True)
    e = jnp.exp(s - m)
    return e / jnp.sum(e, axis=-1, keepdims=True)


def _diag_scale(dcol, A):
    C = A.shape[0]
    DM = _eye(C) * dcol
    return _dot(DM, _dot(A, DM))


def _fix_adj(oa):
    C = oa.shape[0]
    oa = oa * (1.0 - _eye(C))
    dsum = jnp.sum(oa, axis=-1, keepdims=True)
    dsafe = jnp.where(dsum > 0, dsum, 1.0)
    d = jnp.where(dsum > 0, jnp.sqrt(dsafe), 0.0) + _EPS
    return _diag_scale(1.0 / d, oa)


def _pool(x, adj, s):
    sig = _softmax(s)
    out = _tdot(sig, x)
    t = _dot(adj, sig)
    out_adj = _tdot(sig, t)
    num = jnp.sum(out_adj * _eye(out_adj.shape[0]))
    dflat = jnp.sum(adj, axis=-1, keepdims=True)
    den = jnp.sum(dflat * jnp.sum(sig * sig, axis=-1, keepdims=True))
    mc = -(num / den)
    C = sig.shape[-1]
    ss = _tdot(sig, sig)
    ss_norm = jnp.sqrt(jnp.sum(ss * ss))
    diff = ss / ss_norm - _eye(C) / jnp.sqrt(jnp.float32(C))
    sq = jnp.sum(diff * diff)
    ot = jnp.where(sq > 0, jnp.sqrt(jnp.where(sq > 0, sq, 1.0)), 0.0)
    return out, _fix_adj(out_adj), mc, ot


def _gcn_dense(x, adj, W, b):
    C = adj.shape[0]
    A = adj + _eye(C)
    deg = jnp.sum(A, axis=-1, keepdims=True)
    dsafe = jnp.where(deg > 0, deg, 1.0)
    dinv = jnp.where(deg > 0, lax.rsqrt(dsafe), 0.0)
    An = _diag_scale(dinv, A)
    return _dot(An, _dot(x, W)) + b


def _deg_col(degp):
    ones = jnp.ones((degp.shape[0], 1), jnp.float32)
    return _tdot(degp, ones) + 1.0


def _tc_h_body(x_ref, w1_ref, degp_ref, h_ref, gm_ref):
    h = _dot(x_ref[...], w1_ref[...])
    h_ref[...] = h
    gm_ref[...] = lax.rsqrt(_deg_col(degp_ref[...])) * h


def _tc_d_body(accp_ref, degp_ref, h_ref, b1_ref, ltw1_ref, ltb1_ref,
               x1_ref, s1_ref, sigl_ref, sigr_ref, out1_ref, r1_ref, ot1_ref):
    dinv = lax.rsqrt(_deg_col(degp_ref[...]))
    acc = accp_ref[0] + accp_ref[1]
    h = h_ref[...]
    x1 = jnp.maximum(dinv * acc + dinv * dinv * h + b1_ref[...], 0.0)
    s1 = jnp.maximum(_dot(x1, ltw1_ref[...]) + ltb1_ref[...], 0.0)
    sig = _softmax(s1)
    x1_ref[...] = x1
    s1_ref[...] = s1
    sigl_ref[...] = sig[:, :128]
    sigr_ref[...] = sig[:, 128:]
    out1_ref[...] = _tdot(sig, x1)
    r1_ref[...] = jnp.sum(sig * sig, axis=-1, keepdims=True)
    ss1 = _tdot(sig, sig)
    ss_norm1 = jnp.sqrt(jnp.sum(ss1 * ss1))
    diff1 = ss1 / ss_norm1 - _eye(256) / jnp.sqrt(jnp.float32(256))
    sq1 = jnp.sum(diff1 * diff1)
    ot1 = jnp.where(sq1 > 0, jnp.sqrt(jnp.where(sq1 > 0, sq1, 1.0)), 0.0)
    ot1_ref[...] = jnp.broadcast_to(ot1, (1, 1))


def _tc_f_body(sigl_ref, sigr_ref, tpl_ref, tpr_ref, out1_ref, r1_ref,
               w2_ref, b2_ref, ltw2_ref, ltb2_ref,
               w3_ref, b3_ref, ltw3_ref, ltb3_ref,
               x3b_ref, adj3_ref, s2_ref, s3_ref,
               mc1_ref, mc2_ref, mc3_ref, ot2_ref, ot3_ref):
    sig1 = jnp.concatenate([sigl_ref[...], sigr_ref[...]], axis=-1)
    t = jnp.concatenate([tpl_ref[0] + tpl_ref[1], tpr_ref[0] + tpr_ref[1]],
                        axis=-1)

    out_adj1 = _tdot(sig1, t)
    out1 = out1_ref[...]
    num1 = jnp.sum(out_adj1 * _eye(256))
    dflat1 = jnp.sum(t, axis=-1, keepdims=True)
    den1 = jnp.sum(dflat1 * r1_ref[...])
    mc1 = -(num1 / den1)
    adj1 = _fix_adj(out_adj1)

    x2 = jnp.maximum(_gcn_dense(out1, adj1, w2_ref[...], b2_ref[...]), 0.0)
    s2 = jnp.maximum(_dot(x2, ltw2_ref[...]) + ltb2_ref[...], 0.0)
    x2b, adj2, mc2, ot2 = _pool(x2, adj1, s2)

    x3 = jnp.maximum(_gcn_dense(x2b, adj2, w3_ref[...], b3_ref[...]), 0.0)
    s3 = jnp.maximum(_dot(x3, ltw3_ref[...]) + ltb3_ref[...], 0.0)
    x3b, adj3, mc3, ot3 = _pool(x3, adj2, s3)

    x3b_ref[...] = x3b
    adj3_ref[...] = adj3
    s2_ref[...] = s2
    s3_ref[...] = s3
    mc1_ref[...] = jnp.broadcast_to(mc1, (1, 1))
    mc2_ref[...] = jnp.broadcast_to(mc2, (1, 1))
    mc3_ref[...] = jnp.broadcast_to(mc3, (1, 1))
    ot2_ref[...] = jnp.broadcast_to(ot2, (1, 1))
    ot3_ref[...] = jnp.broadcast_to(ot3, (1, 1))


def kernel(x, edge_index1, edge_attr1, W1, b1, ltW1, ltb1,
           W2, b2, ltW2, ltb2, W3, b3, ltW3, ltb3):
    f32 = jnp.float32
    src = edge_index1[0].astype(jnp.int32)
    dst = edge_index1[1].astype(jnp.int32)
    src2 = src.reshape(_E // _B, _B)
    dst2 = dst.reshape(_E // _B, _B)
    wb = jnp.broadcast_to(edge_attr1[:, None], (_E, _L)).astype(f32)
    b1r, b2r, b3r = (b.reshape(1, -1) for b in (b1, b2, b3))
    ltb1r, ltb2r, ltb3r = (b.reshape(1, -1) for b in (ltb1, ltb2, ltb3))

    degp = _make_sc_deg()(dst2, edge_attr1.astype(f32))

    h, gm = pl.pallas_call(
        _tc_h_body,
        out_shape=[jax.ShapeDtypeStruct((_N, 128), f32)] * 2,
    )(x, W1, degp)

    accp = _make_sc_segsum()(gm, src2, dst2, wb)

    x1, s1, sigl, sigr, out1, r1, ot1 = pl.pallas_call(
        _tc_d_body,
        out_shape=[jax.ShapeDtypeStruct((_N, 128), f32),
                   jax.ShapeDtypeStruct((_N, 256), f32),
                   jax.ShapeDtypeStruct((_N, 128), f32),
                   jax.ShapeDtypeStruct((_N, 128), f32),
                   jax.ShapeDtypeStruct((256, 128), f32),
                   jax.ShapeDtypeStruct((_N, 1), f32),
                   jax.ShapeDtypeStruct((1, 1), f32)],
    )(accp, degp, h, b1r, ltW1, ltb1r)

    tpl, tpr = _make_sc_tfused()(sigl, sigr, dst2, src2, wb)

    (x3b, adj3, s2, s3, mc1, mc2, mc3, ot2, ot3) = pl.pallas_call(
        _tc_f_body,
        out_shape=[jax.ShapeDtypeStruct((1, 128), f32),
                   jax.ShapeDtypeStruct((1, 1), f32),
                   jax.ShapeDtypeStruct((256, 64), f32),
                   jax.ShapeDtypeStruct((64, 1), f32)]
                  + [jax.ShapeDtypeStruct((1, 1), f32)] * 5,
    )(sigl, sigr, tpl, tpr, out1, r1, W2, b2r, ltW2, ltb2r, W3, b3r, ltW3, ltb3r)

    scalar = lambda a: a.reshape(())
    return (x3b, adj3, (s1, s2, s3),
            (scalar(mc1), scalar(mc2), scalar(mc3)),
            (scalar(ot1), scalar(ot2), scalar(ot3)))


# speedup vs baseline: 1.0410x; 1.0410x over previous
import dataclasses
import functools

import jax
import jax.numpy as jnp
from jax import lax
from jax.experimental import pallas as pl
from jax.experimental.pallas import tpu as pltpu
from jax.experimental.pallas import tpu_sc as plsc

_EPS = 1e-15
_N = 4096
_E = 65536
_NC = 2
_NS = 16
_L = 16
_B = 128
_PER_TILE = _E // (_NC * _NS)
_NBLK = _PER_TILE // _B
_STRIPE = _N // _NS
_ROWS_PER_TILE = _PER_TILE // _B


def _mesh():
    return plsc.VectorSubcoreMesh(core_axis_name="c", subcore_axis_name="s")


def _zero_fill(buf, nrows, D):
    @pl.loop(0, nrows)
    def _(i):
        for k in range(D // _L):
            buf[i, pl.ds(k * _L, _L)] = jnp.zeros((_L,), jnp.float32)


def _init_acc(acc_sh, zbuf, sid, D):
    _zero_fill(zbuf, _B, D)
    for r in range(_STRIPE // _B):
        pltpu.sync_copy(zbuf, acc_sh.at[pl.ds(sid * _STRIPE + r * _B, _B)])


def _drain_acc(acc_sh, out_hbm, cid, sid):
    for r in range(_STRIPE // _B):
        off = sid * _STRIPE + r * _B
        pltpu.sync_copy(acc_sh.at[pl.ds(off, _B)], out_hbm.at[cid, pl.ds(off, _B)])


@functools.cache
def _make_sc_segsum():
    D = 128

    @functools.partial(
        pl.kernel,
        out_type=jax.ShapeDtypeStruct((_NC, _N, D), jnp.float32),
        mesh=_mesh(),
        scratch_types=[
            pltpu.VMEM((_ROWS_PER_TILE, _B), jnp.int32),
            pltpu.VMEM((_ROWS_PER_TILE, _B), jnp.int32),
            pltpu.VMEM((2, _B, _L), jnp.float32),
            pltpu.VMEM((2, _B, D), jnp.float32),
            pltpu.VMEM_SHARED((_N, D), jnp.float32),
            pltpu.SemaphoreType.DMA,
            pltpu.SemaphoreType.DMA,
            pltpu.SemaphoreType.DMA,
            pltpu.SemaphoreType.DMA,
        ],
    )
    def seg(table_hbm, gidx_hbm, sidx_hbm, wb_hbm, out_hbm,
            gidx_v, sidx_v, wb_v, rows_v, acc_sh, g0, g1, s0, s1):
        cid = lax.axis_index("c")
        sid = lax.axis_index("s")
        _init_acc(acc_sh, rows_v.at[0], sid, D)

        tile_base = (cid * _NS + sid) * _PER_TILE
        row_base = (cid * _NS + sid) * _ROWS_PER_TILE
        pltpu.sync_copy(gidx_hbm.at[pl.ds(row_base, _ROWS_PER_TILE)], gidx_v)
        pltpu.sync_copy(sidx_hbm.at[pl.ds(row_base, _ROWS_PER_TILE)], sidx_v)
        plsc.subcore_barrier()

        @pl.loop(0, _NBLK, step=2)
        def _(blk):
            ga = pltpu.async_copy(
                table_hbm.at[gidx_v.at[blk]], rows_v.at[0], g0)
            gb = pltpu.async_copy(
                table_hbm.at[gidx_v.at[blk + 1]], rows_v.at[1], g1)
            pltpu.sync_copy(wb_hbm.at[pl.ds(tile_base + blk * _B, _B)],
                            wb_v.at[0])
            pltpu.sync_copy(wb_hbm.at[pl.ds(tile_base + (blk + 1) * _B, _B)],
                            wb_v.at[1])
            scatters = []
            for b, gwait, ssem in ((0, ga, s0), (1, gb, s1)):
                gwait.wait()

                @plsc.parallel_loop(0, _B, unroll=4)
                def _(j):
                    c = wb_v[b, j, :]
                    for k in range(D // _L):
                        sl = (b, j, pl.ds(k * _L, _L))
                        rows_v[sl] = rows_v[sl] * c

                scatters.append(pltpu.async_copy(
                    rows_v.at[b], acc_sh.at[sidx_v.at[blk + b]], ssem,
                    add=True))
            for sc in scatters:
                sc.wait()

        plsc.subcore_barrier()
        _drain_acc(acc_sh, out_hbm, cid, sid)

    return seg


@functools.cache
def _make_sc_tfused():
    D = 128

    @functools.partial(
        pl.kernel,
        out_type=[jax.ShapeDtypeStruct((_NC, _N, D), jnp.float32)] * 2,
        mesh=_mesh(),
        scratch_types=[
            pltpu.VMEM((_ROWS_PER_TILE, _B), jnp.int32),
            pltpu.VMEM((_ROWS_PER_TILE, _B), jnp.int32),
            pltpu.VMEM((_B, _L), jnp.float32),
            pltpu.VMEM((_B, D), jnp.float32),
            pltpu.VMEM((_B, D), jnp.float32),
            pltpu.VMEM_SHARED((_N, D), jnp.float32),
            pltpu.VMEM_SHARED((_N, D), jnp.float32),
            pltpu.SemaphoreType.DMA,
            pltpu.SemaphoreType.DMA,
        ],
    )
    def seg2(tabl_hbm, tabr_hbm, gidx_hbm, sidx_hbm, wb_hbm, outl_hbm, outr_hbm,
             gidx_v, sidx_v, wb_v, rowsl_v, rowsr_v, accl_sh, accr_sh,
             sl0, sr0):
        cid = lax.axis_index("c")
        sid = lax.axis_index("s")
        _init_acc(accl_sh, rowsl_v, sid, D)
        _init_acc(accr_sh, rowsr_v, sid, D)

        tile_base = (cid * _NS + sid) * _PER_TILE
        row_base = (cid * _NS + sid) * _ROWS_PER_TILE
        pltpu.sync_copy(gidx_hbm.at[pl.ds(row_base, _ROWS_PER_TILE)], gidx_v)
        pltpu.sync_copy(sidx_hbm.at[pl.ds(row_base, _ROWS_PER_TILE)], sidx_v)
        plsc.subcore_barrier()

        @pl.loop(0, _NBLK)
        def _(blk):
            gl = pltpu.async_copy(
                tabl_hbm.at[gidx_v.at[blk]], rowsl_v, sl0)
            gr = pltpu.async_copy(
                tabr_hbm.at[gidx_v.at[blk]], rowsr_v, sr0)
            pltpu.sync_copy(
                wb_hbm.at[pl.ds(tile_base + blk * _B, _B)], wb_v)
            gl.wait()
            gr.wait()

            @plsc.parallel_loop(0, _B, unroll=4)
            def _(j):
                c = wb_v[j, :]
                for k in range(D // _L):
                    sl = (j, pl.ds(k * _L, _L))
                    rowsl_v[sl] = rowsl_v[sl] * c
                    rowsr_v[sl] = rowsr_v[sl] * c

            scl = pltpu.async_copy(
                rowsl_v, accl_sh.at[sidx_v.at[blk]], sl0, add=True)
            scr = pltpu.async_copy(
                rowsr_v, accr_sh.at[sidx_v.at[blk]], sr0, add=True)
            scl.wait()
            scr.wait()

        plsc.subcore_barrier()
        _drain_acc(accl_sh, outl_hbm, cid, sid)
        _drain_acc(accr_sh, outr_hbm, cid, sid)

    return seg2


@functools.cache
def _make_sc_deg():
    cp = pltpu.CompilerParams()
    if "needs_layout_passes" in pltpu.CompilerParams.__dataclass_fields__:
        cp = dataclasses.replace(cp, needs_layout_passes=False)

    @functools.partial(
        pl.kernel,
        out_type=jax.ShapeDtypeStruct((_NC * _NS, _N), jnp.float32),
        mesh=_mesh(),
        scratch_types=[
            pltpu.VMEM((_ROWS_PER_TILE, _B), jnp.int32),
            pltpu.VMEM((_PER_TILE,), jnp.float32),
            pltpu.VMEM((_N,), jnp.float32),
        ],
        compiler_params=cp,
    )
    def degk(sidx_hbm, w_hbm, out_hbm, sidx_v, w_v, acc_v):
        cid = lax.axis_index("c")
        sid = lax.axis_index("s")
        wid = cid * _NS + sid

        for i in range(_N // _L):
            acc_v[pl.ds(i * _L, _L)] = jnp.zeros((_L,), jnp.float32)

        tile_base = wid * _PER_TILE
        row_base = wid * _ROWS_PER_TILE
        pltpu.sync_copy(sidx_hbm.at[pl.ds(row_base, _ROWS_PER_TILE)], sidx_v)
        pltpu.sync_copy(w_hbm.at[pl.ds(tile_base, _PER_TILE)], w_v)

        for blk in range(_NBLK):
            for g in range(_B // _L):
                idx16 = sidx_v[blk, pl.ds(g * _L, _L)]
                w16 = w_v[pl.ds((blk * (_B // _L) + g) * _L, _L)]
                plsc.addupdate_scatter(acc_v, [idx16], w16)

        pltpu.sync_copy(acc_v, out_hbm.at[wid])

    return degk


def _eye(C, dtype=jnp.float32):
    r = lax.broadcasted_iota(jnp.int32, (C, C), 0)
    c = lax.broadcasted_iota(jnp.int32, (C, C), 1)
    return jnp.where(r == c, 1.0, 0.0).astype(dtype)


def _tdot(a, b):
    return lax.dot_general(a, b, dimension_numbers=(((0,), (0,)), ((), ())),
                           preferred_element_type=jnp.float32)


def _dot(a, b):
    return jnp.dot(a, b, preferred_element_type=jnp.float32)


def _softmax(s):
    m = jnp.max(s, axis=-1, keepdims=True)
    e = jnp.exp(s - m)
    return e / jnp.sum(e, axis=-1, keepdims=True)


def _diag_scale(dcol, A):
    C = A.shape[0]
    DM = _eye(C) * dcol
    return _dot(DM, _dot(A, DM))


def _fix_adj(oa):
    C = oa.shape[0]
    oa = oa * (1.0 - _eye(C))
    dsum = jnp.sum(oa, axis=-1, keepdims=True)
    dsafe = jnp.where(dsum > 0, dsum, 1.0)
    d = jnp.where(dsum > 0, jnp.sqrt(dsafe), 0.0) + _EPS
    return _diag_scale(1.0 / d, oa)


def _pool(x, adj, s):
    sig = _softmax(s)
    out = _tdot(sig, x)
    t = _dot(adj, sig)
    out_adj = _tdot(sig, t)
    num = jnp.sum(out_adj * _eye(out_adj.shape[0]))
    dflat = jnp.sum(adj, axis=-1, keepdims=True)
    den = jnp.sum(dflat * jnp.sum(sig * sig, axis=-1, keepdims=True))
    mc = -(num / den)
    C = sig.shape[-1]
    ss = _tdot(sig, sig)
    ss_norm = jnp.sqrt(jnp.sum(ss * ss))
    diff = ss / ss_norm - _eye(C) / jnp.sqrt(jnp.float32(C))
    sq = jnp.sum(diff * diff)
    ot = jnp.where(sq > 0, jnp.sqrt(jnp.where(sq > 0, sq, 1.0)), 0.0)
    return out, _fix_adj(out_adj), mc, ot


def _gcn_dense(x, adj, W, b):
    C = adj.shape[0]
    A = adj + _eye(C)
    deg = jnp.sum(A, axis=-1, keepdims=True)
    dsafe = jnp.where(deg > 0, deg, 1.0)
    dinv = jnp.where(deg > 0, lax.rsqrt(dsafe), 0.0)
    An = _diag_scale(dinv, A)
    return _dot(An, _dot(x, W)) + b


def _deg_col(degp):
    ones = jnp.ones((degp.shape[0], 1), jnp.float32)
    return _tdot(degp, ones) + 1.0


def _tc_h_body(x_ref, w1_ref, degp_ref, h_ref, gm_ref):
    h = _dot(x_ref[...], w1_ref[...])
    h_ref[...] = h
    gm_ref[...] = lax.rsqrt(_deg_col(degp_ref[...])) * h


def _tc_d_body(accp_ref, degp_ref, h_ref, b1_ref, ltw1_ref, ltb1_ref,
               x1_ref, s1_ref, sigl_ref, sigr_ref):
    dinv = lax.rsqrt(_deg_col(degp_ref[...]))
    acc = accp_ref[0] + accp_ref[1]
    h = h_ref[...]
    x1 = jnp.maximum(dinv * acc + dinv * dinv * h + b1_ref[...], 0.0)
    s1 = jnp.maximum(_dot(x1, ltw1_ref[...]) + ltb1_ref[...], 0.0)
    sig = _softmax(s1)
    x1_ref[...] = x1
    s1_ref[...] = s1
    sigl_ref[...] = sig[:, :128]
    sigr_ref[...] = sig[:, 128:]


def _tc_f1_body(x1_ref, sigl_ref, sigr_ref,
                out1_ref, r1_ref, ot1_ref):
    sig1 = jnp.concatenate([sigl_ref[...], sigr_ref[...]], axis=-1)
    out1 = _tdot(sig1, x1_ref[...])
    r1 = jnp.sum(sig1 * sig1, axis=-1, keepdims=True)
    ss1 = _tdot(sig1, sig1)
    ss_norm1 = jnp.sqrt(jnp.sum(ss1 * ss1))
    diff1 = ss1 / ss_norm1 - _eye(256) / jnp.sqrt(jnp.float32(256))
    sq1 = jnp.sum(diff1 * diff1)
    ot1 = jnp.where(sq1 > 0, jnp.sqrt(jnp.where(sq1 > 0, sq1, 1.0)), 0.0)
    out1_ref[...] = out1
    r1_ref[...] = r1
    ot1_ref[...] = jnp.broadcast_to(ot1, (1, 1))


def _tc_f_body(sigl_ref, sigr_ref, tpl_ref, tpr_ref, out1_ref, r1_ref,
               w2_ref, b2_ref, ltw2_ref, ltb2_ref,
               w3_ref, b3_ref, ltw3_ref, ltb3_ref,
               x3b_ref, adj3_ref, s2_ref, s3_ref,
               mc1_ref, mc2_ref, mc3_ref, ot2_ref, ot3_ref):
    sig1 = jnp.concatenate([sigl_ref[...], sigr_ref[...]], axis=-1)
    t = jnp.concatenate([tpl_ref[0] + tpl_ref[1], tpr_ref[0] + tpr_ref[1]],
                        axis=-1)

    out_adj1 = _tdot(sig1, t)
    out1 = out1_ref[...]
    num1 = jnp.sum(out_adj1 * _eye(256))
    dflat1 = jnp.sum(t, axis=-1, keepdims=True)
    den1 = jnp.sum(dflat1 * r1_ref[...])
    mc1 = -(num1 / den1)
    adj1 = _fix_adj(out_adj1)

    x2 = jnp.maximum(_gcn_dense(out1, adj1, w2_ref[...], b2_ref[...]), 0.0)
    s2 = jnp.maximum(_dot(x2, ltw2_ref[...]) + ltb2_ref[...], 0.0)
    x2b, adj2, mc2, ot2 = _pool(x2, adj1, s2)

    x3 = jnp.maximum(_gcn_dense(x2b, adj2, w3_ref[...], b3_ref[...]), 0.0)
    s3 = jnp.maximum(_dot(x3, ltw3_ref[...]) + ltb3_ref[...], 0.0)
    x3b, adj3, mc3, ot3 = _pool(x3, adj2, s3)

    x3b_ref[...] = x3b
    adj3_ref[...] = adj3
    s2_ref[...] = s2
    s3_ref[...] = s3
    mc1_ref[...] = jnp.broadcast_to(mc1, (1, 1))
    mc2_ref[...] = jnp.broadcast_to(mc2, (1, 1))
    mc3_ref[...] = jnp.broadcast_to(mc3, (1, 1))
    ot2_ref[...] = jnp.broadcast_to(ot2, (1, 1))
    ot3_ref[...] = jnp.broadcast_to(ot3, (1, 1))


def kernel(x, edge_index1, edge_attr1, W1, b1, ltW1, ltb1,
           W2, b2, ltW2, ltb2, W3, b3, ltW3, ltb3):
    f32 = jnp.float32
    src = edge_index1[0].astype(jnp.int32)
    dst = edge_index1[1].astype(jnp.int32)
    src2 = src.reshape(_E // _B, _B)
    dst2 = dst.reshape(_E // _B, _B)
    wb = jnp.broadcast_to(edge_attr1[:, None], (_E, _L)).astype(f32)
    b1r, b2r, b3r = (b.reshape(1, -1) for b in (b1, b2, b3))
    ltb1r, ltb2r, ltb3r = (b.reshape(1, -1) for b in (ltb1, ltb2, ltb3))

    degp = _make_sc_deg()(dst2, edge_attr1.astype(f32))

    h, gm = pl.pallas_call(
        _tc_h_body,
        out_shape=[jax.ShapeDtypeStruct((_N, 128), f32)] * 2,
    )(x, W1, degp)

    accp = _make_sc_segsum()(gm, src2, dst2, wb)

    x1, s1, sigl, sigr = pl.pallas_call(
        _tc_d_body,
        out_shape=[jax.ShapeDtypeStruct((_N, 128), f32),
                   jax.ShapeDtypeStruct((_N, 256), f32),
                   jax.ShapeDtypeStruct((_N, 128), f32),
                   jax.ShapeDtypeStruct((_N, 128), f32)],
    )(accp, degp, h, b1r, ltW1, ltb1r)

    tpl, tpr = _make_sc_tfused()(sigl, sigr, dst2, src2, wb)

    out1, r1, ot1 = pl.pallas_call(
        _tc_f1_body,
        out_shape=[jax.ShapeDtypeStruct((256, 128), f32),
                   jax.ShapeDtypeStruct((_N, 1), f32),
                   jax.ShapeDtypeStruct((1, 1), f32)],
    )(x1, sigl, sigr)

    (x3b, adj3, s2, s3, mc1, mc2, mc3, ot2, ot3) = pl.pallas_call(
        _tc_f_body,
        out_shape=[jax.ShapeDtypeStruct((1, 128), f32),
                   jax.ShapeDtypeStruct((1, 1), f32),
                   jax.ShapeDtypeStruct((256, 64), f32),
                   jax.ShapeDtypeStruct((64, 1), f32)]
                  + [jax.ShapeDtypeStruct((1, 1), f32)] * 5,
    )(sigl, sigr, tpl, tpr, out1, r1, W2, b2r, ltW2, ltb2r, W3, b3r, ltW3, ltb3r)

    scalar = lambda a: a.reshape(())
    return (x3b, adj3, (s1, s2, s3),
            (scalar(mc1), scalar(mc2), scalar(mc3)),
            (scalar(ot1), scalar(ot2), scalar(ot3)))
